# baseline (speedup 1.0000x reference)
import functools

import jax
import jax.numpy as jnp
from jax import lax
from jax.experimental import pallas as pl
from jax.experimental.pallas import tpu as pltpu
from jax.experimental.pallas import tpu_sc as plsc

BATCH = 1024
STATE_DIM = 768
LATENT_DIM = 256
NUM_EMB = 1024
HIDDEN = 64

_HI = lax.Precision.HIGHEST
_BB = 256
_NBLK = BATCH // _BB

def _dn(c_lhs, c_rhs):
    return (((c_lhs,), (c_rhs,)), ((), ()))


def _tc_body(x_ref, w1_ref, b1_ref, w2_ref, b2_ref, emb_ref, w3_ref, b3_ref,
             w4_ref, b4_ref, z_e_ref, idx_ref, tbl_ref):
    i = pl.program_id(0)
    h = jnp.maximum(
        lax.dot_general(x_ref[...], w1_ref[...], _dn(1, 0)) + b1_ref[...], 0.0)
    z_e = lax.dot_general(h, w2_ref[...], _dn(1, 0)) + b2_ref[...]
    z_e_ref[...] = z_e
    emb = emb_ref[...]
    cross = lax.dot_general(z_e, emb, _dn(1, 1), precision=_HI)
    ones = jnp.ones((1, LATENT_DIM), jnp.float32)
    norms = lax.dot_general(ones, emb * emb, _dn(1, 1), precision=_HI)
    scores = norms - 2.0 * cross
    m = jnp.min(scores, axis=1, keepdims=True)
    iota = lax.broadcasted_iota(jnp.int32, (_BB, NUM_EMB), 1)
    idx_ref[...] = jnp.min(
        jnp.where(scores <= m, iota, NUM_EMB), axis=1, keepdims=True
    )
    e_blk = emb_ref[pl.ds(i * _BB, _BB), :]
    h2 = jnp.maximum(
        lax.dot_general(e_blk, w3_ref[...], _dn(1, 0)) + b3_ref[...], 0.0)
    tbl_ref[...] = lax.dot_general(h2, w4_ref[...], _dn(1, 0)) + b4_ref[...]


_NC = 2
_NS = 16
_NW = _NC * _NS
_BPW = BATCH // _NW


def _make_sc_gather():
    mesh = plsc.VectorSubcoreMesh(core_axis_name="c", subcore_axis_name="s")

    @functools.partial(
        pl.kernel,
        mesh=mesh,
        out_type=(
            jax.ShapeDtypeStruct((BATCH, LATENT_DIM), jnp.float32),
            jax.ShapeDtypeStruct((BATCH, STATE_DIM), jnp.float32),
        ),
        scratch_types=[
            pltpu.VMEM((_BPW,), jnp.int32),
            pltpu.VMEM((_BPW, LATENT_DIM), jnp.float32),
            pltpu.VMEM((_BPW, STATE_DIM), jnp.float32),
            pltpu.SemaphoreType.DMA,
            pltpu.SemaphoreType.DMA,
        ],
    )
    def _sc_gather(emb_hbm, tbl_hbm, idx_hbm, zq_hbm, xr_hbm,
                   idx_v, zq_v, xr_v, sem1, sem2):
        wid = lax.axis_index("s") * _NC + lax.axis_index("c")
        base = wid * _BPW
        pltpu.sync_copy(idx_hbm.at[pl.ds(base, _BPW)], idx_v)
        cp1 = pltpu.async_copy(emb_hbm.at[idx_v], zq_v, sem1)
        cp2 = pltpu.async_copy(tbl_hbm.at[idx_v], xr_v, sem2)
        cp1.wait()
        pltpu.sync_copy(zq_v, zq_hbm.at[pl.ds(base, _BPW)])
        cp2.wait()
        pltpu.sync_copy(xr_v, xr_hbm.at[pl.ds(base, _BPW)])

    return _sc_gather


def kernel(x, W1, b1, W2, b2, emb, W3, b3, W4, b4):
    z_e, idx2, tbl = pl.pallas_call(
        _tc_body,
        grid=(_NBLK,),
        in_specs=[
            pl.BlockSpec((_BB, STATE_DIM), lambda i: (i, 0)),
            pl.BlockSpec((STATE_DIM, HIDDEN), lambda i: (0, 0)),
            pl.BlockSpec((1, HIDDEN), lambda i: (0, 0)),
            pl.BlockSpec((HIDDEN, LATENT_DIM), lambda i: (0, 0)),
            pl.BlockSpec((1, LATENT_DIM), lambda i: (0, 0)),
            pl.BlockSpec((NUM_EMB, LATENT_DIM), lambda i: (0, 0)),
            pl.BlockSpec((LATENT_DIM, HIDDEN), lambda i: (0, 0)),
            pl.BlockSpec((1, HIDDEN), lambda i: (0, 0)),
            pl.BlockSpec((HIDDEN, STATE_DIM), lambda i: (0, 0)),
            pl.BlockSpec((1, STATE_DIM), lambda i: (0, 0)),
        ],
        out_specs=[
            pl.BlockSpec((_BB, LATENT_DIM), lambda i: (i, 0)),
            pl.BlockSpec((_BB, 1), lambda i: (i, 0)),
            pl.BlockSpec((_BB, STATE_DIM), lambda i: (i, 0)),
        ],
        out_shape=[
            jax.ShapeDtypeStruct((BATCH, LATENT_DIM), jnp.float32),
            jax.ShapeDtypeStruct((BATCH, 1), jnp.int32),
            jax.ShapeDtypeStruct((NUM_EMB, STATE_DIM), jnp.float32),
        ],
    )(x, W1, b1.reshape(1, HIDDEN), W2, b2.reshape(1, LATENT_DIM), emb,
      W3, b3.reshape(1, HIDDEN), W4, b4.reshape(1, STATE_DIM))
    z_q, x_recon = _make_sc_gather()(emb, tbl, idx2.reshape(BATCH))
    return (x_recon, z_e, z_q)
